# Initial kernel scaffold; baseline (speedup 1.0000x reference)
import functools

import jax
import jax.numpy as jnp
from jax import lax
from jax.experimental import pallas as pl
from jax.experimental.pallas import tpu as pltpu
from jax.experimental.pallas import tpu_sc as plsc

_N = 10000
_D = 128
_E = 320000
_NW = 32
_EPW = _E // _NW
_NCHUNK = 80
_K = _EPW // _NCHUNK
_RPS = _N // 16
_ZR = 125

_sc_mesh = plsc.VectorSubcoreMesh(core_axis_name="c", subcore_axis_name="s")


@functools.partial(
    pl.kernel,
    out_type=jax.ShapeDtypeStruct((2, _N, _D), jnp.float32),
    mesh=_sc_mesh,
    scratch_types=[
        pltpu.VMEM((_NCHUNK, _K), jnp.int32),
        pltpu.VMEM((_NCHUNK, _K), jnp.int32),
        pltpu.VMEM((_K, _D), jnp.float32),
        pltpu.VMEM((_ZR, _D), jnp.float32),
        pltpu.VMEM_SHARED((_N, _D), jnp.float32),
        pltpu.SemaphoreType.DMA,
    ],
)
def _segment_sum_sc(h_hbm, src_hbm, dst_hbm, out_hbm,
                    src_v, dst_v, rows_v, zero_v, agg_sh, sem):
    cid = lax.axis_index("c")
    sid = lax.axis_index("s")
    w = cid * 16 + sid

    def _zrow(r, carry):
        def _zcol(c, carry2):
            zero_v[r, pl.ds(c * 16, 16)] = jnp.zeros((16,), jnp.float32)
            return carry2
        return lax.fori_loop(0, _D // 16, _zcol, carry)
    lax.fori_loop(0, _ZR, _zrow, None)

    for t in range(_RPS // _ZR):
        pltpu.sync_copy(zero_v, agg_sh.at[pl.ds(sid * _RPS + t * _ZR, _ZR)])
    plsc.subcore_barrier()

    pltpu.sync_copy(src_hbm.at[w], src_v)
    pltpu.sync_copy(dst_hbm.at[w], dst_v)

    def _chunk(j, carry):
        pltpu.async_copy(h_hbm.at[src_v.at[j]], rows_v, sem).wait()
        pltpu.sync_copy(rows_v, agg_sh.at[dst_v.at[j]], add=True)
        return carry
    lax.fori_loop(0, _NCHUNK, _chunk, None)

    plsc.subcore_barrier()
    pltpu.sync_copy(agg_sh.at[pl.ds(sid * _RPS, _RPS)],
                    out_hbm.at[cid, pl.ds(sid * _RPS, _RPS)])


_RB = 1000


def _make_mlp(relu_out, last):
    def _body(h_ref, a0_ref, a1_ref, w1_ref, b1_ref, w2_ref, b2_ref,
              wm_ref, pin_ref, bm_ref, hout_ref, pout_ref):
        z = h_ref[...] + a0_ref[...] + a1_ref[...]
        t = jnp.dot(z, w1_ref[...], preferred_element_type=jnp.float32)
        t = jnp.maximum(t + b1_ref[...], 0.0)
        u = jnp.dot(t, w2_ref[...], preferred_element_type=jnp.float32)
        u = u + b2_ref[...]
        if relu_out:
            u = jnp.maximum(u, 0.0)
        hout_ref[...] = u
        p = pin_ref[...] + jnp.sum(u * wm_ref[...], axis=1, keepdims=True)
        if last:
            p = jax.nn.sigmoid(p + bm_ref[...])
        pout_ref[...] = p
    return _body


def _mlp_call(body):
    row_spec = pl.BlockSpec((_RB, _D), lambda i: (i, 0))
    vec_spec = pl.BlockSpec((1, _D), lambda i: (0, 0))
    mat_spec = pl.BlockSpec((_D, _D), lambda i: (0, 0))
    p_spec = pl.BlockSpec((_RB, 1), lambda i: (i, 0))
    one_spec = pl.BlockSpec((1, 1), lambda i: (0, 0))
    return pl.pallas_call(
        body,
        grid=(_N // _RB,),
        in_specs=[row_spec, row_spec, row_spec, mat_spec, vec_spec,
                  mat_spec, vec_spec, vec_spec, p_spec, one_spec],
        out_specs=[row_spec, p_spec],
        out_shape=[jax.ShapeDtypeStruct((_N, _D), jnp.float32),
                   jax.ShapeDtypeStruct((_N, 1), jnp.float32)],
        compiler_params=pltpu.CompilerParams(
            dimension_semantics=("parallel",)),
    )


_mlp_mid = _mlp_call(_make_mlp(relu_out=True, last=False))
_mlp_last = _mlp_call(_make_mlp(relu_out=False, last=True))


def kernel(x, edge_index, batch, W1_0, b1_0, W2_0, b2_0, W1_1, b1_1, W2_1,
           b2_1, W1_2, b1_2, W2_2, b2_2, Wm, bm):
    src = edge_index[0].astype(jnp.int32).reshape(_NW, _NCHUNK, _K)
    dst = edge_index[1].astype(jnp.int32).reshape(_NW, _NCHUNK, _K)
    params = [(W1_0, b1_0, W2_0, b2_0), (W1_1, b1_1, W2_1, b2_1),
              (W1_2, b1_2, W2_2, b2_2)]
    wm = Wm.astype(jnp.float32).reshape(3, 1, _D)
    bm2 = bm.astype(jnp.float32).reshape(1, 1)

    h = x
    p = jnp.zeros((_N, 1), jnp.float32)
    for i in range(3):
        W1, b1, W2, b2 = params[i]
        parts = _segment_sum_sc(h, src, dst)
        call = _mlp_last if i == 2 else _mlp_mid
        h, p = call(h, parts[0], parts[1], W1, b1.reshape(1, _D),
                    W2, b2.reshape(1, _D), wm[i], p, bm2)
    return p


# speedup vs baseline: 5.4974x; 5.4974x over previous
import functools

import jax
import jax.numpy as jnp
from jax import lax
from jax.experimental import pallas as pl
from jax.experimental.pallas import tpu as pltpu
from jax.experimental.pallas import tpu_sc as plsc

_N = 10000
_D = 128
_H = _D // 2
_E = 320000
_NSUB = 16
_EPW = _E // _NSUB
_NCHUNK = 160
_K = _EPW // _NCHUNK
_NP = 10240
_RPS = _NP // _NSUB
_ZR = 128

_sc_mesh = plsc.VectorSubcoreMesh(core_axis_name="c", subcore_axis_name="s")


@functools.partial(
    pl.kernel,
    out_type=jax.ShapeDtypeStruct((2, _NP, _H), jnp.float32),
    mesh=_sc_mesh,
    scratch_types=[
        pltpu.VMEM((_NCHUNK, _K), jnp.int32),
        pltpu.VMEM((_NCHUNK, _K), jnp.int32),
        pltpu.VMEM((_K, _H), jnp.float32),
        pltpu.VMEM((_ZR, _H), jnp.float32),
        pltpu.VMEM_SHARED((_NP, _H), jnp.float32),
        pltpu.SemaphoreType.DMA,
    ],
    compiler_params=pltpu.CompilerParams(use_tc_tiling_on_sc=False),
)
def _segment_sum_sc(h2_hbm, src_hbm, dst_hbm, out_hbm,
                    src_v, dst_v, rows_v, zero_v, agg_sh, sem):
    cid = lax.axis_index("c")
    sid = lax.axis_index("s")

    def _zrow(r, carry):
        def _zcol(c, carry2):
            zero_v[r, pl.ds(c * 16, 16)] = jnp.zeros((16,), jnp.float32)
            return carry2
        return lax.fori_loop(0, _H // 16, _zcol, carry)
    lax.fori_loop(0, _ZR, _zrow, None)

    for t in range(_RPS // _ZR):
        pltpu.sync_copy(zero_v, agg_sh.at[pl.ds(sid * _RPS + t * _ZR, _ZR)])
    plsc.subcore_barrier()

    pltpu.sync_copy(src_hbm.at[sid], src_v)
    pltpu.sync_copy(dst_hbm.at[sid], dst_v)

    def _chunk(j, carry):
        pltpu.async_copy(h2_hbm.at[cid].at[src_v.at[j]], rows_v, sem).wait()
        pltpu.sync_copy(rows_v, agg_sh.at[dst_v.at[j]], add=True)
        return carry
    lax.fori_loop(0, _NCHUNK, _chunk, None)

    plsc.subcore_barrier()
    pltpu.sync_copy(agg_sh.at[pl.ds(sid * _RPS, _RPS)],
                    out_hbm.at[cid, pl.ds(sid * _RPS, _RPS)])


_RB = 1000


def _mlp_body(h_ref, a_ref, w1_ref, b1_ref, w2_ref, b2_ref,
              wm_ref, pin_ref, bm_ref, hout_ref, pout_ref, *,
              relu_out, last):
    z = jnp.concatenate([h_ref[0] + a_ref[0], h_ref[1] + a_ref[1]], axis=1)
    t = jnp.dot(z, w1_ref[...], preferred_element_type=jnp.float32)
    t = jnp.maximum(t + b1_ref[...], 0.0)
    u = jnp.dot(t, w2_ref[...], preferred_element_type=jnp.float32)
    u = u + b2_ref[...]
    if relu_out:
        u = jnp.maximum(u, 0.0)
    p = pin_ref[...] + jnp.sum(u * wm_ref[...], axis=1, keepdims=True)
    if last:
        p = jax.nn.sigmoid(p + bm_ref[...])
    else:
        hout_ref[0] = u[:, :_H]
        hout_ref[1] = u[:, _H:]
    pout_ref[...] = p


def _mid_body(h_ref, a_ref, w1_ref, b1_ref, w2_ref, b2_ref,
              wm_ref, pin_ref, bm_ref, hout_ref, pout_ref):
    _mlp_body(h_ref, a_ref, w1_ref, b1_ref, w2_ref, b2_ref, wm_ref,
              pin_ref, bm_ref, hout_ref, pout_ref, relu_out=True, last=False)


def _last_body(h_ref, a_ref, w1_ref, b1_ref, w2_ref, b2_ref,
               wm_ref, pin_ref, bm_ref, pout_ref):
    _mlp_body(h_ref, a_ref, w1_ref, b1_ref, w2_ref, b2_ref, wm_ref,
              pin_ref, bm_ref, None, pout_ref, relu_out=False, last=True)


def _mlp_call(body, last):
    half_spec = pl.BlockSpec((2, _RB, _H), lambda i: (0, i, 0))
    vec_spec = pl.BlockSpec((1, _D), lambda i: (0, 0))
    mat_spec = pl.BlockSpec((_D, _D), lambda i: (0, 0))
    p_spec = pl.BlockSpec((_RB, 1), lambda i: (i, 0))
    one_spec = pl.BlockSpec((1, 1), lambda i: (0, 0))
    if last:
        out_specs = [p_spec]
        out_shape = [jax.ShapeDtypeStruct((_N, 1), jnp.float32)]
    else:
        out_specs = [half_spec, p_spec]
        out_shape = [jax.ShapeDtypeStruct((2, _N, _H), jnp.float32),
                     jax.ShapeDtypeStruct((_N, 1), jnp.float32)]
    return pl.pallas_call(
        body,
        grid=(_N // _RB,),
        in_specs=[half_spec, half_spec, mat_spec, vec_spec,
                  mat_spec, vec_spec, vec_spec, p_spec, one_spec],
        out_specs=out_specs,
        out_shape=out_shape,
        compiler_params=pltpu.CompilerParams(
            dimension_semantics=("parallel",)),
    )


_mlp_mid = _mlp_call(_mid_body, last=False)
_mlp_last = _mlp_call(_last_body, last=True)


def kernel(x, edge_index, batch, W1_0, b1_0, W2_0, b2_0, W1_1, b1_1, W2_1,
           b2_1, W1_2, b1_2, W2_2, b2_2, Wm, bm):
    src = edge_index[0].astype(jnp.int32).reshape(_NSUB, _NCHUNK, _K)
    dst = edge_index[1].astype(jnp.int32).reshape(_NSUB, _NCHUNK, _K)
    params = [(W1_0, b1_0, W2_0, b2_0), (W1_1, b1_1, W2_1, b2_1),
              (W1_2, b1_2, W2_2, b2_2)]
    wm = Wm.astype(jnp.float32).reshape(3, 1, _D)
    bm2 = bm.astype(jnp.float32).reshape(1, 1)

    h2 = x.reshape(_N, 2, _H).transpose(1, 0, 2)
    p = jnp.zeros((_N, 1), jnp.float32)
    for i in range(3):
        W1, b1, W2, b2 = params[i]
        agg2 = _segment_sum_sc(h2, src, dst)
        if i < 2:
            h2, p = _mlp_mid(h2, agg2, W1, b1.reshape(1, _D),
                             W2, b2.reshape(1, _D), wm[i], p, bm2)
        else:
            (p,) = _mlp_last(h2, agg2, W1, b1.reshape(1, _D),
                             W2, b2.reshape(1, _D), wm[i], p, bm2)
    return p


# speedup vs baseline: 8.5381x; 1.5531x over previous
import functools

import jax
import jax.numpy as jnp
from jax import lax
from jax.experimental import pallas as pl
from jax.experimental.pallas import tpu as pltpu
from jax.experimental.pallas import tpu_sc as plsc

_N = 10000
_D = 128
_H = _D // 2
_E = 320000
_NSUB = 16
_EPW = _E // _NSUB
_NCHUNK = 160
_K = _EPW // _NCHUNK
_NP = 10240
_RPS = _NP // _NSUB
_ZR = 128

_sc_mesh = plsc.VectorSubcoreMesh(core_axis_name="c", subcore_axis_name="s")


@functools.partial(
    pl.kernel,
    out_type=jax.ShapeDtypeStruct((2, _NP, _H), jnp.float32),
    mesh=_sc_mesh,
    scratch_types=[
        pltpu.VMEM((_NCHUNK, _K), jnp.int32),
        pltpu.VMEM((_NCHUNK, _K), jnp.int32),
        pltpu.VMEM((_K, _H), jnp.float32),
        pltpu.VMEM((_K, _H), jnp.float32),
        pltpu.VMEM((_ZR, _H), jnp.float32),
        pltpu.VMEM_SHARED((_NP, _H), jnp.float32),
        pltpu.SemaphoreType.DMA,
        pltpu.SemaphoreType.DMA,
    ],
    compiler_params=pltpu.CompilerParams(use_tc_tiling_on_sc=False),
)
def _segment_sum_sc(h2_hbm, src_hbm, dst_hbm, out_hbm,
                    src_v, dst_v, rows0_v, rows1_v, zero_v, agg_sh,
                    sem0, sem1):
    cid = lax.axis_index("c")
    sid = lax.axis_index("s")

    def _zrow(r, carry):
        def _zcol(c, carry2):
            zero_v[r, pl.ds(c * 16, 16)] = jnp.zeros((16,), jnp.float32)
            return carry2
        return lax.fori_loop(0, _H // 16, _zcol, carry)
    lax.fori_loop(0, _ZR, _zrow, None)

    for t in range(_RPS // _ZR):
        pltpu.sync_copy(zero_v, agg_sh.at[pl.ds(sid * _RPS + t * _ZR, _ZR)])
    plsc.subcore_barrier()

    pltpu.sync_copy(src_hbm.at[sid], src_v)
    pltpu.sync_copy(dst_hbm.at[sid], dst_v)

    bufs = ((rows0_v, sem0), (rows1_v, sem1))
    pltpu.async_copy(h2_hbm.at[cid].at[src_v.at[0]], rows0_v, sem0)
    pltpu.async_copy(h2_hbm.at[cid].at[src_v.at[1]], rows1_v, sem1)

    def _chunk(t, carry):
        for b in range(2):
            j = 2 * t + b
            rows_v, sem = bufs[b]
            pltpu.make_async_copy(h2_hbm.at[cid].at[src_v.at[j]],
                                  rows_v, sem).wait()
            pltpu.sync_copy(rows_v, agg_sh.at[dst_v.at[j]], add=True)

            @pl.when(j + 2 < _NCHUNK)
            def _prefetch():
                pltpu.async_copy(h2_hbm.at[cid].at[src_v.at[j + 2]],
                                 rows_v, sem)
        return carry
    lax.fori_loop(0, _NCHUNK // 2, _chunk, None)

    plsc.subcore_barrier()
    pltpu.sync_copy(agg_sh.at[pl.ds(sid * _RPS, _RPS)],
                    out_hbm.at[cid, pl.ds(sid * _RPS, _RPS)])


_RB = 1000


def _mlp_body(h_ref, a_ref, w1_ref, b1_ref, w2_ref, b2_ref,
              wm_ref, pin_ref, bm_ref, hout_ref, pout_ref, *,
              relu_out, last):
    z = jnp.concatenate([h_ref[0] + a_ref[0], h_ref[1] + a_ref[1]], axis=1)
    t = jnp.dot(z, w1_ref[...], preferred_element_type=jnp.float32)
    t = jnp.maximum(t + b1_ref[...], 0.0)
    u = jnp.dot(t, w2_ref[...], preferred_element_type=jnp.float32)
    u = u + b2_ref[...]
    if relu_out:
        u = jnp.maximum(u, 0.0)
    p = pin_ref[...] + jnp.sum(u * wm_ref[...], axis=1, keepdims=True)
    if last:
        p = jax.nn.sigmoid(p + bm_ref[...])
    else:
        hout_ref[0] = u[:, :_H]
        hout_ref[1] = u[:, _H:]
    pout_ref[...] = p


def _mid_body(h_ref, a_ref, w1_ref, b1_ref, w2_ref, b2_ref,
              wm_ref, pin_ref, bm_ref, hout_ref, pout_ref):
    _mlp_body(h_ref, a_ref, w1_ref, b1_ref, w2_ref, b2_ref, wm_ref,
              pin_ref, bm_ref, hout_ref, pout_ref, relu_out=True, last=False)


def _last_body(h_ref, a_ref, w1_ref, b1_ref, w2_ref, b2_ref,
               wm_ref, pin_ref, bm_ref, pout_ref):
    _mlp_body(h_ref, a_ref, w1_ref, b1_ref, w2_ref, b2_ref, wm_ref,
              pin_ref, bm_ref, None, pout_ref, relu_out=False, last=True)


def _mlp_call(body, last):
    half_spec = pl.BlockSpec((2, _RB, _H), lambda i: (0, i, 0))
    vec_spec = pl.BlockSpec((1, _D), lambda i: (0, 0))
    mat_spec = pl.BlockSpec((_D, _D), lambda i: (0, 0))
    p_spec = pl.BlockSpec((_RB, 1), lambda i: (i, 0))
    one_spec = pl.BlockSpec((1, 1), lambda i: (0, 0))
    if last:
        out_specs = [p_spec]
        out_shape = [jax.ShapeDtypeStruct((_N, 1), jnp.float32)]
    else:
        out_specs = [half_spec, p_spec]
        out_shape = [jax.ShapeDtypeStruct((2, _N, _H), jnp.float32),
                     jax.ShapeDtypeStruct((_N, 1), jnp.float32)]
    return pl.pallas_call(
        body,
        grid=(_N // _RB,),
        in_specs=[half_spec, half_spec, mat_spec, vec_spec,
                  mat_spec, vec_spec, vec_spec, p_spec, one_spec],
        out_specs=out_specs,
        out_shape=out_shape,
        compiler_params=pltpu.CompilerParams(
            dimension_semantics=("parallel",)),
    )


_mlp_mid = _mlp_call(_mid_body, last=False)
_mlp_last = _mlp_call(_last_body, last=True)


def kernel(x, edge_index, batch, W1_0, b1_0, W2_0, b2_0, W1_1, b1_1, W2_1,
           b2_1, W1_2, b1_2, W2_2, b2_2, Wm, bm):
    src = edge_index[0].astype(jnp.int32).reshape(_NSUB, _NCHUNK, _K)
    dst = edge_index[1].astype(jnp.int32).reshape(_NSUB, _NCHUNK, _K)
    params = [(W1_0, b1_0, W2_0, b2_0), (W1_1, b1_1, W2_1, b2_1),
              (W1_2, b1_2, W2_2, b2_2)]
    wm = Wm.astype(jnp.float32).reshape(3, 1, _D)
    bm2 = bm.astype(jnp.float32).reshape(1, 1)

    h2 = x.reshape(_N, 2, _H).transpose(1, 0, 2)
    p = jnp.zeros((_N, 1), jnp.float32)
    for i in range(3):
        W1, b1, W2, b2 = params[i]
        agg2 = _segment_sum_sc(h2, src, dst)
        if i < 2:
            h2, p = _mlp_mid(h2, agg2, W1, b1.reshape(1, _D),
                             W2, b2.reshape(1, _D), wm[i], p, bm2)
        else:
            (p,) = _mlp_last(h2, agg2, W1, b1.reshape(1, _D),
                             W2, b2.reshape(1, _D), wm[i], p, bm2)
    return p


# speedup vs baseline: 8.8976x; 1.0421x over previous
import functools

import jax
import jax.numpy as jnp
from jax import lax
from jax.experimental import pallas as pl
from jax.experimental.pallas import tpu as pltpu
from jax.experimental.pallas import tpu_sc as plsc

_N = 10000
_D = 128
_H = _D // 2
_E = 320000
_NSUB = 16
_EPW = _E // _NSUB
_NCHUNK = 160
_K = _EPW // _NCHUNK
_NP = 10240
_RPS = _NP // _NSUB
_ZR = 128

_sc_mesh = plsc.VectorSubcoreMesh(core_axis_name="c", subcore_axis_name="s")


@functools.partial(
    pl.kernel,
    out_type=jax.ShapeDtypeStruct((2, _NP, _H), jnp.float32),
    mesh=_sc_mesh,
    scratch_types=[
        pltpu.VMEM((_NCHUNK, _K), jnp.int32),
        pltpu.VMEM((_NCHUNK, _K), jnp.int32),
        [pltpu.VMEM((_K, _H), jnp.float32)] * 4,
        pltpu.VMEM((_ZR, _H), jnp.float32),
        pltpu.VMEM_SHARED((_NP, _H), jnp.float32),
        [pltpu.SemaphoreType.DMA] * 4,
        [pltpu.SemaphoreType.DMA] * 4,
    ],
    compiler_params=pltpu.CompilerParams(use_tc_tiling_on_sc=False),
)
def _segment_sum_sc(h2_hbm, src_hbm, dst_hbm, out_hbm,
                    src_v, dst_v, rows, zero_v, agg_sh, gsem, ssem):
    cid = lax.axis_index("c")
    sid = lax.axis_index("s")

    def _zrow(r, carry):
        def _zcol(c, carry2):
            zero_v[r, pl.ds(c * 16, 16)] = jnp.zeros((16,), jnp.float32)
            return carry2
        return lax.fori_loop(0, _H // 16, _zcol, carry)
    lax.fori_loop(0, _ZR, _zrow, None)

    for t in range(_RPS // _ZR):
        pltpu.sync_copy(zero_v, agg_sh.at[pl.ds(sid * _RPS + t * _ZR, _ZR)])
    plsc.subcore_barrier()

    pltpu.sync_copy(src_hbm.at[sid], src_v)
    pltpu.sync_copy(dst_hbm.at[sid], dst_v)

    pltpu.async_copy(h2_hbm.at[cid].at[src_v.at[0]], rows[0], gsem[0])
    pltpu.async_copy(h2_hbm.at[cid].at[src_v.at[1]], rows[1], gsem[1])

    def _chunk(t, carry):
        for b in range(4):
            j = 4 * t + b
            pltpu.make_async_copy(h2_hbm.at[cid].at[src_v.at[j]],
                                  rows[b], gsem[b]).wait()
            pltpu.async_copy(rows[b], agg_sh.at[dst_v.at[j]], ssem[b],
                             add=True)
            bp = (b + 2) % 4

            @pl.when(j >= 2)
            def _drain():
                pltpu.make_async_copy(
                    rows[bp], agg_sh.at[dst_v.at[j - 2]], ssem[bp]).wait()

            @pl.when(j + 2 < _NCHUNK)
            def _prefetch():
                pltpu.async_copy(h2_hbm.at[cid].at[src_v.at[j + 2]],
                                 rows[bp], gsem[bp])
        return carry
    lax.fori_loop(0, _NCHUNK // 4, _chunk, None)
    pltpu.make_async_copy(rows[2], agg_sh.at[dst_v.at[_NCHUNK - 2]],
                          ssem[2]).wait()
    pltpu.make_async_copy(rows[3], agg_sh.at[dst_v.at[_NCHUNK - 1]],
                          ssem[3]).wait()

    plsc.subcore_barrier()
    pltpu.sync_copy(agg_sh.at[pl.ds(sid * _RPS, _RPS)],
                    out_hbm.at[cid, pl.ds(sid * _RPS, _RPS)])


_RB = 1000


def _mlp_body(h_ref, a_ref, w1_ref, b1_ref, w2_ref, b2_ref,
              wm_ref, pin_ref, bm_ref, hout_ref, pout_ref, *,
              relu_out, last):
    z = jnp.concatenate([h_ref[0] + a_ref[0], h_ref[1] + a_ref[1]], axis=1)
    t = jnp.dot(z, w1_ref[...], preferred_element_type=jnp.float32)
    t = jnp.maximum(t + b1_ref[...], 0.0)
    u = jnp.dot(t, w2_ref[...], preferred_element_type=jnp.float32)
    u = u + b2_ref[...]
    if relu_out:
        u = jnp.maximum(u, 0.0)
    p = pin_ref[...] + jnp.sum(u * wm_ref[...], axis=1, keepdims=True)
    if last:
        p = jax.nn.sigmoid(p + bm_ref[...])
    else:
        hout_ref[0] = u[:, :_H]
        hout_ref[1] = u[:, _H:]
    pout_ref[...] = p


def _mid_body(h_ref, a_ref, w1_ref, b1_ref, w2_ref, b2_ref,
              wm_ref, pin_ref, bm_ref, hout_ref, pout_ref):
    _mlp_body(h_ref, a_ref, w1_ref, b1_ref, w2_ref, b2_ref, wm_ref,
              pin_ref, bm_ref, hout_ref, pout_ref, relu_out=True, last=False)


def _last_body(h_ref, a_ref, w1_ref, b1_ref, w2_ref, b2_ref,
               wm_ref, pin_ref, bm_ref, pout_ref):
    _mlp_body(h_ref, a_ref, w1_ref, b1_ref, w2_ref, b2_ref, wm_ref,
              pin_ref, bm_ref, None, pout_ref, relu_out=False, last=True)


def _mlp_call(body, last):
    half_spec = pl.BlockSpec((2, _RB, _H), lambda i: (0, i, 0))
    vec_spec = pl.BlockSpec((1, _D), lambda i: (0, 0))
    mat_spec = pl.BlockSpec((_D, _D), lambda i: (0, 0))
    p_spec = pl.BlockSpec((_RB, 1), lambda i: (i, 0))
    one_spec = pl.BlockSpec((1, 1), lambda i: (0, 0))
    if last:
        out_specs = [p_spec]
        out_shape = [jax.ShapeDtypeStruct((_N, 1), jnp.float32)]
    else:
        out_specs = [half_spec, p_spec]
        out_shape = [jax.ShapeDtypeStruct((2, _N, _H), jnp.float32),
                     jax.ShapeDtypeStruct((_N, 1), jnp.float32)]
    return pl.pallas_call(
        body,
        grid=(_N // _RB,),
        in_specs=[half_spec, half_spec, mat_spec, vec_spec,
                  mat_spec, vec_spec, vec_spec, p_spec, one_spec],
        out_specs=out_specs,
        out_shape=out_shape,
        compiler_params=pltpu.CompilerParams(
            dimension_semantics=("parallel",)),
    )


_mlp_mid = _mlp_call(_mid_body, last=False)
_mlp_last = _mlp_call(_last_body, last=True)


def kernel(x, edge_index, batch, W1_0, b1_0, W2_0, b2_0, W1_1, b1_1, W2_1,
           b2_1, W1_2, b1_2, W2_2, b2_2, Wm, bm):
    src = edge_index[0].astype(jnp.int32).reshape(_NSUB, _NCHUNK, _K)
    dst = edge_index[1].astype(jnp.int32).reshape(_NSUB, _NCHUNK, _K)
    params = [(W1_0, b1_0, W2_0, b2_0), (W1_1, b1_1, W2_1, b2_1),
              (W1_2, b1_2, W2_2, b2_2)]
    wm = Wm.astype(jnp.float32).reshape(3, 1, _D)
    bm2 = bm.astype(jnp.float32).reshape(1, 1)

    h2 = x.reshape(_N, 2, _H).transpose(1, 0, 2)
    p = jnp.zeros((_N, 1), jnp.float32)
    for i in range(3):
        W1, b1, W2, b2 = params[i]
        agg2 = _segment_sum_sc(h2, src, dst)
        if i < 2:
            h2, p = _mlp_mid(h2, agg2, W1, b1.reshape(1, _D),
                             W2, b2.reshape(1, _D), wm[i], p, bm2)
        else:
            (p,) = _mlp_last(h2, agg2, W1, b1.reshape(1, _D),
                             W2, b2.reshape(1, _D), wm[i], p, bm2)
    return p


# speedup vs baseline: 9.9600x; 1.1194x over previous
import functools

import jax
import jax.numpy as jnp
from jax import lax
from jax.experimental import pallas as pl
from jax.experimental.pallas import tpu as pltpu
from jax.experimental.pallas import tpu_sc as plsc

_N = 10000
_D = 128
_H = _D // 2
_E = 320000
_NSUB = 16
_EPW = _E // _NSUB
_NCHUNK = 160
_K = _EPW // _NCHUNK
_NP = 10240
_RPS = _NP // _NSUB
_ZR = 128

_sc_mesh = plsc.VectorSubcoreMesh(core_axis_name="c", subcore_axis_name="s")


@functools.partial(
    pl.kernel,
    out_type=jax.ShapeDtypeStruct((2, _NP, _H), jnp.float32),
    mesh=_sc_mesh,
    scratch_types=[
        pltpu.VMEM((_NCHUNK, _K), jnp.int32),
        pltpu.VMEM((_NCHUNK, _K), jnp.int32),
        [pltpu.VMEM((_K, _H), jnp.float32)] * 3,
        pltpu.VMEM((_ZR, _H), jnp.float32),
        pltpu.VMEM_SHARED((_NP, _H), jnp.float32),
        [pltpu.SemaphoreType.DMA] * 3,
        pltpu.SemaphoreType.DMA,
    ],
    compiler_params=pltpu.CompilerParams(use_tc_tiling_on_sc=False),
)
def _segment_sum_sc(h2_hbm, eidx_hbm, out_hbm,
                    src_v, dst_v, rows, zero_v, agg_sh, gsem, ssem):
    cid = lax.axis_index("c")
    sid = lax.axis_index("s")

    def _zrow(r, carry):
        def _zcol(c, carry2):
            zero_v[r, pl.ds(c * 16, 16)] = jnp.zeros((16,), jnp.float32)
            return carry2
        return lax.fori_loop(0, _H // 16, _zcol, carry)
    lax.fori_loop(0, _ZR, _zrow, None)

    for t in range(_RPS // _ZR):
        pltpu.sync_copy(zero_v, agg_sh.at[pl.ds(sid * _RPS + t * _ZR, _ZR)])
    plsc.subcore_barrier()

    pltpu.sync_copy(eidx_hbm.at[0, sid], src_v)
    pltpu.sync_copy(eidx_hbm.at[1, sid], dst_v)

    pltpu.async_copy(h2_hbm.at[cid].at[src_v.at[0]], rows[0], gsem[0])
    pltpu.async_copy(h2_hbm.at[cid].at[src_v.at[1]], rows[1], gsem[1])

    def _chunk(t, carry):
        for b in range(3):
            j = 3 * t + b
            pltpu.make_async_copy(h2_hbm.at[cid].at[src_v.at[j]],
                                  rows[b], gsem[b]).wait()

            @pl.when(j >= 1)
            def _drain():
                bp = (b + 2) % 3
                pltpu.make_async_copy(
                    rows[bp], agg_sh.at[dst_v.at[j - 1]], ssem).wait()
            pltpu.async_copy(rows[b], agg_sh.at[dst_v.at[j]], ssem,
                             add=True)

            @pl.when(j + 2 < _NCHUNK)
            def _prefetch():
                bn = (b + 2) % 3
                pltpu.async_copy(h2_hbm.at[cid].at[src_v.at[j + 2]],
                                 rows[bn], gsem[bn])
        return carry
    lax.fori_loop(0, _NCHUNK // 3, _chunk, None)

    j_last = _NCHUNK - 1
    b_last = j_last % 3
    pltpu.make_async_copy(h2_hbm.at[cid].at[src_v.at[j_last]],
                          rows[b_last], gsem[b_last]).wait()
    pltpu.make_async_copy(rows[(b_last + 2) % 3],
                          agg_sh.at[dst_v.at[j_last - 1]], ssem).wait()
    pltpu.async_copy(rows[b_last], agg_sh.at[dst_v.at[j_last]], ssem,
                     add=True)
    pltpu.make_async_copy(rows[b_last], agg_sh.at[dst_v.at[j_last]],
                          ssem).wait()

    plsc.subcore_barrier()
    pltpu.sync_copy(agg_sh.at[pl.ds(sid * _RPS, _RPS)],
                    out_hbm.at[cid, pl.ds(sid * _RPS, _RPS)])


_RB = 1000


def _mlp_body(h_ref, a_ref, w1_ref, b1_ref, w2_ref, b2_ref,
              wm_ref, pin_ref, bm_ref, hout_ref, pout_ref, *,
              relu_out, last):
    z = jnp.concatenate([h_ref[0] + a_ref[0], h_ref[1] + a_ref[1]], axis=1)
    t = jnp.dot(z, w1_ref[...], preferred_element_type=jnp.float32)
    t = jnp.maximum(t + b1_ref[...], 0.0)
    u = jnp.dot(t, w2_ref[...], preferred_element_type=jnp.float32)
    u = u + b2_ref[...]
    if relu_out:
        u = jnp.maximum(u, 0.0)
    p = pin_ref[...] + jnp.sum(u * wm_ref[...], axis=1, keepdims=True)
    if last:
        p = jax.nn.sigmoid(p + bm_ref[...])
    else:
        hout_ref[0] = u[:, :_H]
        hout_ref[1] = u[:, _H:]
    pout_ref[...] = p


def _mid_body(h_ref, a_ref, w1_ref, b1_ref, w2_ref, b2_ref,
              wm_ref, pin_ref, bm_ref, hout_ref, pout_ref):
    _mlp_body(h_ref, a_ref, w1_ref, b1_ref, w2_ref, b2_ref, wm_ref,
              pin_ref, bm_ref, hout_ref, pout_ref, relu_out=True, last=False)


def _last_body(h_ref, a_ref, w1_ref, b1_ref, w2_ref, b2_ref,
               wm_ref, pin_ref, bm_ref, pout_ref):
    _mlp_body(h_ref, a_ref, w1_ref, b1_ref, w2_ref, b2_ref, wm_ref,
              pin_ref, bm_ref, None, pout_ref, relu_out=False, last=True)


def _mlp_call(body, last):
    half_spec = pl.BlockSpec((2, _RB, _H), lambda i: (0, i, 0))
    vec_spec = pl.BlockSpec((1, _D), lambda i: (0, 0))
    mat_spec = pl.BlockSpec((_D, _D), lambda i: (0, 0))
    p_spec = pl.BlockSpec((_RB, 1), lambda i: (i, 0))
    one_spec = pl.BlockSpec((1, 1), lambda i: (0, 0))
    if last:
        out_specs = [p_spec]
        out_shape = [jax.ShapeDtypeStruct((_N, 1), jnp.float32)]
    else:
        out_specs = [half_spec, p_spec]
        out_shape = [jax.ShapeDtypeStruct((2, _N, _H), jnp.float32),
                     jax.ShapeDtypeStruct((_N, 1), jnp.float32)]
    return pl.pallas_call(
        body,
        grid=(_N // _RB,),
        in_specs=[half_spec, half_spec, mat_spec, vec_spec,
                  mat_spec, vec_spec, vec_spec, p_spec, one_spec],
        out_specs=out_specs,
        out_shape=out_shape,
        compiler_params=pltpu.CompilerParams(
            dimension_semantics=("parallel",)),
    )


_mlp_mid = _mlp_call(_mid_body, last=False)
_mlp_last = _mlp_call(_last_body, last=True)


def kernel(x, edge_index, batch, W1_0, b1_0, W2_0, b2_0, W1_1, b1_1, W2_1,
           b2_1, W1_2, b1_2, W2_2, b2_2, Wm, bm):
    eidx = edge_index.astype(jnp.int32).reshape(2, _NSUB, _NCHUNK, _K)
    params = [(W1_0, b1_0, W2_0, b2_0), (W1_1, b1_1, W2_1, b2_1),
              (W1_2, b1_2, W2_2, b2_2)]
    wm = Wm.astype(jnp.float32).reshape(3, 1, _D)
    bm2 = bm.astype(jnp.float32).reshape(1, 1)

    h2 = x.reshape(_N, 2, _H).transpose(1, 0, 2)
    p = jnp.zeros((_N, 1), jnp.float32)
    for i in range(3):
        W1, b1, W2, b2 = params[i]
        agg2 = _segment_sum_sc(h2, eidx)
        if i < 2:
            h2, p = _mlp_mid(h2, agg2, W1, b1.reshape(1, _D),
                             W2, b2.reshape(1, _D), wm[i], p, bm2)
        else:
            (p,) = _mlp_last(h2, agg2, W1, b1.reshape(1, _D),
                             W2, b2.reshape(1, _D), wm[i], p, bm2)
    return p
